# baseline (speedup 1.0000x reference)
import functools

import jax
import jax.numpy as jnp
from jax import lax
from jax.experimental import pallas as pl
from jax.experimental.pallas import tpu as pltpu
from jax.experimental.pallas import tpu_sc as plsc


def kernel(pos_id, pos_table):
    t = pos_id.shape[1]
    d = pos_table.shape[1]

    mesh = plsc.ScalarSubcoreMesh(axis_name="c", num_cores=2)
    rows_per_c = t // mesh.num_cores
    assert t % mesh.num_cores == 0

    @functools.partial(
        pl.kernel,
        out_type=jax.ShapeDtypeStruct((t, d), pos_table.dtype),
        mesh=mesh,
        scratch_types=[pltpu.SemaphoreType.DMA],
    )
    def copy_rows(table_hbm, out_hbm, sem):
        cid = lax.axis_index("c")
        base = cid * rows_per_c
        pltpu.async_copy(
            table_hbm.at[pl.ds(base, rows_per_c)],
            out_hbm.at[pl.ds(base, rows_per_c)],
            sem,
        ).wait()

    return copy_rows(pos_table)


# speedup vs baseline: 16.0875x; 16.0875x over previous
import functools

import jax
import jax.numpy as jnp
from jax import lax
from jax.experimental import pallas as pl
from jax.experimental.pallas import tpu as pltpu
from jax.experimental.pallas import tpu_sc as plsc

_NBUF = 4


def kernel(pos_id, pos_table):
    t = pos_id.shape[1]
    d = pos_table.shape[1]

    mesh = plsc.ScalarSubcoreMesh(axis_name="c", num_cores=2)
    rows_per_c = t // mesh.num_cores
    assert t % mesh.num_cores == 0
    chunk = 256
    assert rows_per_c % chunk == 0
    nchunks = rows_per_c // chunk

    @functools.partial(
        pl.kernel,
        out_type=jax.ShapeDtypeStruct((t, d), pos_table.dtype),
        mesh=mesh,
        scratch_types=(
            [pltpu.VMEM_SHARED((_NBUF, chunk, d), pos_table.dtype)]
            + [pltpu.SemaphoreType.DMA] * (2 * _NBUF)
        ),
    )
    def copy_rows(table_hbm, out_hbm, buf, *sems):
        sem_in, sem_out = sems[:_NBUF], sems[_NBUF:]
        cid = lax.axis_index("c")
        base = cid * rows_per_c

        def in_copy(g, b):
            return pltpu.make_async_copy(
                table_hbm.at[pl.ds(base + g * chunk, chunk)], buf.at[b], sem_in[b]
            )

        def out_copy(g, b):
            return pltpu.make_async_copy(
                buf.at[b], out_hbm.at[pl.ds(base + g * chunk, chunk)], sem_out[b]
            )

        for b in range(min(_NBUF, nchunks)):
            in_copy(b, b).start()
        for g in range(nchunks):
            b = g % _NBUF
            in_copy(g, b).wait()
            out_copy(g, b).start()
            if g + _NBUF < nchunks:
                out_copy(g, b).wait()
                in_copy(g + _NBUF, b).start()
        for g in range(max(0, nchunks - _NBUF), nchunks):
            out_copy(g, g % _NBUF).wait()

    return copy_rows(pos_table)


# speedup vs baseline: 16.6482x; 1.0349x over previous
import functools

import jax
import jax.numpy as jnp
from jax import lax
from jax.experimental import pallas as pl
from jax.experimental.pallas import tpu as pltpu
from jax.experimental.pallas import tpu_sc as plsc

_NBUF = 2


def kernel(pos_id, pos_table):
    t = pos_id.shape[1]
    d = pos_table.shape[1]

    mesh = plsc.VectorSubcoreMesh(core_axis_name="c", subcore_axis_name="s")
    nw = mesh.num_cores * mesh.num_subcores
    assert t % nw == 0
    rows_per_w = t // nw
    chunk = 32
    assert rows_per_w % chunk == 0
    nchunks = rows_per_w // chunk

    @functools.partial(
        pl.kernel,
        out_type=jax.ShapeDtypeStruct((t, d), pos_table.dtype),
        mesh=mesh,
        scratch_types=(
            [pltpu.VMEM((_NBUF, chunk, d), pos_table.dtype)]
            + [pltpu.SemaphoreType.DMA] * (2 * _NBUF)
        ),
    )
    def copy_rows(table_hbm, out_hbm, buf, *sems):
        sem_in, sem_out = sems[:_NBUF], sems[_NBUF:]
        wid = lax.axis_index("s") * mesh.num_cores + lax.axis_index("c")
        base = wid * rows_per_w

        def in_copy(g, b):
            return pltpu.make_async_copy(
                table_hbm.at[pl.ds(base + g * chunk, chunk)], buf.at[b], sem_in[b]
            )

        def out_copy(g, b):
            return pltpu.make_async_copy(
                buf.at[b], out_hbm.at[pl.ds(base + g * chunk, chunk)], sem_out[b]
            )

        for b in range(min(_NBUF, nchunks)):
            in_copy(b, b).start()
        for g in range(nchunks):
            b = g % _NBUF
            in_copy(g, b).wait()
            out_copy(g, b).start()
            if g + _NBUF < nchunks:
                out_copy(g, b).wait()
                in_copy(g + _NBUF, b).start()
        for g in range(max(0, nchunks - _NBUF), nchunks):
            out_copy(g, g % _NBUF).wait()

    return copy_rows(pos_table)


# speedup vs baseline: 17.6100x; 1.0578x over previous
import functools

import jax
import jax.numpy as jnp
from jax import lax
from jax.experimental import pallas as pl
from jax.experimental.pallas import tpu as pltpu
from jax.experimental.pallas import tpu_sc as plsc

_NBUF = 6


def kernel(pos_id, pos_table):
    t = pos_id.shape[1]
    d = pos_table.shape[1]

    mesh = plsc.VectorSubcoreMesh(core_axis_name="c", subcore_axis_name="s")
    nw = mesh.num_cores * mesh.num_subcores
    assert t % nw == 0
    rows_per_w = t // nw
    chunk = 16
    assert rows_per_w % chunk == 0
    nchunks = rows_per_w // chunk

    @functools.partial(
        pl.kernel,
        out_type=jax.ShapeDtypeStruct((t, d), pos_table.dtype),
        mesh=mesh,
        scratch_types=(
            [pltpu.VMEM((_NBUF, chunk, d), pos_table.dtype)]
            + [pltpu.SemaphoreType.DMA] * (2 * _NBUF)
        ),
    )
    def copy_rows(table_hbm, out_hbm, buf, *sems):
        sem_in, sem_out = sems[:_NBUF], sems[_NBUF:]
        wid = lax.axis_index("s") * mesh.num_cores + lax.axis_index("c")
        base = wid * rows_per_w

        def in_copy(g, b):
            return pltpu.make_async_copy(
                table_hbm.at[pl.ds(base + g * chunk, chunk)], buf.at[b], sem_in[b]
            )

        def out_copy(g, b):
            return pltpu.make_async_copy(
                buf.at[b], out_hbm.at[pl.ds(base + g * chunk, chunk)], sem_out[b]
            )

        for b in range(min(_NBUF, nchunks)):
            in_copy(b, b).start()
        for g in range(nchunks):
            b = g % _NBUF
            in_copy(g, b).wait()
            out_copy(g, b).start()
            if g + _NBUF < nchunks:
                out_copy(g, b).wait()
                in_copy(g + _NBUF, b).start()
        for g in range(max(0, nchunks - _NBUF), nchunks):
            out_copy(g, g % _NBUF).wait()

    return copy_rows(pos_table)
